# Initial kernel scaffold; baseline (speedup 1.0000x reference)
import jax
import jax.numpy as jnp
from jax.experimental import pallas as pl


def kernel(x, edge_index, W1, b1, W2, b2):
    raise NotImplementedError("write your pallas kernel here")


# speedup vs baseline: 12.2933x; 12.2933x over previous
import functools

import jax
import jax.numpy as jnp
from jax import lax
from jax.experimental import pallas as pl
from jax.experimental.pallas import tpu as pltpu
from jax.experimental.pallas import tpu_sc as plsc

N = 10000
E = 320000
F_IN = 128
HID = 128
C = 64
K = 10
ALPHA = 0.1

NC = 2
NS = 16
NW = NC * NS
CHUNK = 128
NCHUNK = E // CHUNK
FULL = NCHUNK // NW
EXTRA = NCHUNK - FULL * NW
N2 = 10240
RPS = N2 // NS
ZR = 128

_SC_MESH = plsc.VectorSubcoreMesh(core_axis_name="c", subcore_axis_name="s")
_SC_PARAMS = pltpu.CompilerParams(use_tc_tiling_on_sc=False)


_MLP_BLK = 1000


def _mlp_body(x_ref, w1_ref, b1_ref, w2_ref, b2_ref, o_ref):
    h = jnp.dot(x_ref[...], w1_ref[...], preferred_element_type=jnp.float32)
    h = jnp.maximum(h + b1_ref[...], 0.0)
    o_ref[...] = (
        jnp.dot(h, w2_ref[...], preferred_element_type=jnp.float32) + b2_ref[...]
    )


def _mlp(x, W1, b1, W2, b2):
    grid = (N // _MLP_BLK,)
    return pl.pallas_call(
        _mlp_body,
        grid=grid,
        in_specs=[
            pl.BlockSpec((_MLP_BLK, F_IN), lambda i: (i, 0)),
            pl.BlockSpec((F_IN, HID), lambda i: (0, 0)),
            pl.BlockSpec((1, HID), lambda i: (0, 0)),
            pl.BlockSpec((HID, C), lambda i: (0, 0)),
            pl.BlockSpec((1, C), lambda i: (0, 0)),
        ],
        out_specs=pl.BlockSpec((_MLP_BLK, C), lambda i: (i, 0)),
        out_shape=jax.ShapeDtypeStruct((N, C), jnp.float32),
    )(x, W1, b1.reshape(1, HID), W2, b2.reshape(1, C))


_ROW_BLK = 1000


def _prep_body(dp_ref, h0_ref, dinv_ref, hs_ref):
    deg = dp_ref[0, :, 0:1] + dp_ref[1, :, 0:1] + 1.0
    dinv = lax.rsqrt(deg)
    dinv_ref[...] = dinv
    hs_ref[...] = dinv * h0_ref[...]


def _prep(dp, h0):
    grid = (N // _ROW_BLK,)
    return pl.pallas_call(
        _prep_body,
        grid=grid,
        in_specs=[
            pl.BlockSpec((2, _ROW_BLK, 16), lambda i: (0, i, 0)),
            pl.BlockSpec((_ROW_BLK, C), lambda i: (i, 0)),
        ],
        out_specs=[
            pl.BlockSpec((_ROW_BLK, 1), lambda i: (i, 0)),
            pl.BlockSpec((_ROW_BLK, C), lambda i: (i, 0)),
        ],
        out_shape=[
            jax.ShapeDtypeStruct((N, 1), jnp.float32),
            jax.ShapeDtypeStruct((N, C), jnp.float32),
        ],
    )(dp, h0)


def _combine_body(p_ref, hs_ref, h0_ref, dinv_ref, hn_ref, hsn_ref):
    dinv = dinv_ref[...]
    agg = dinv * (p_ref[0] + p_ref[1] + hs_ref[...])
    hn = (1.0 - ALPHA) * agg + ALPHA * h0_ref[...]
    hn_ref[...] = hn
    hsn_ref[...] = dinv * hn


def _combine(p, hs, h0, dinv):
    grid = (N // _ROW_BLK,)
    return pl.pallas_call(
        _combine_body,
        grid=grid,
        in_specs=[
            pl.BlockSpec((2, _ROW_BLK, C), lambda i: (0, i, 0)),
            pl.BlockSpec((_ROW_BLK, C), lambda i: (i, 0)),
            pl.BlockSpec((_ROW_BLK, C), lambda i: (i, 0)),
            pl.BlockSpec((_ROW_BLK, 1), lambda i: (i, 0)),
        ],
        out_specs=[
            pl.BlockSpec((_ROW_BLK, C), lambda i: (i, 0)),
            pl.BlockSpec((_ROW_BLK, C), lambda i: (i, 0)),
        ],
        out_shape=[
            jax.ShapeDtypeStruct((N, C), jnp.float32),
            jax.ShapeDtypeStruct((N, C), jnp.float32),
        ],
    )(p, hs, h0, dinv)


def _deg_sc(dst2d):
    @functools.partial(
        pl.kernel,
        out_type=jax.ShapeDtypeStruct((NC, N2, 16), jnp.float32),
        mesh=_SC_MESH,
        compiler_params=_SC_PARAMS,
        scratch_types=[
            pltpu.VMEM_SHARED((N2, 16), jnp.float32),
            pltpu.VMEM((1, CHUNK), jnp.int32),
            pltpu.VMEM((CHUNK, 16), jnp.float32),
            pltpu.VMEM((RPS, 16), jnp.float32),
        ],
    )
    def k(dst_hbm, out_hbm, dacc, dst_v, ones_v, zer_v):
        cid = lax.axis_index("c")
        sid = lax.axis_index("s")
        wid = sid * NC + cid

        @pl.loop(0, CHUNK)
        def _(r):
            ones_v.at[pl.ds(r, 1), pl.ds(0, 16)][...] = jnp.ones((1, 16), jnp.float32)

        @pl.loop(0, RPS)
        def _(r):
            zer_v.at[pl.ds(r, 1), pl.ds(0, 16)][...] = jnp.zeros((1, 16), jnp.float32)

        pltpu.sync_copy(zer_v, dacc.at[pl.ds(sid * RPS, RPS)])
        plsc.subcore_barrier()

        def chunk(r):
            pltpu.sync_copy(dst_hbm.at[r], dst_v.at[0])
            pltpu.sync_copy(ones_v, dacc.at[dst_v.at[0]], add=True)

        @pl.loop(0, FULL)
        def _(j):
            chunk(wid + j * NW)

        @pl.when(wid < EXTRA)
        def _():
            chunk(wid + FULL * NW)

        plsc.subcore_barrier()
        pltpu.sync_copy(
            dacc.at[pl.ds(sid * RPS, RPS)],
            out_hbm.at[cid, pl.ds(sid * RPS, RPS)],
        )

    return k(dst2d)


def _prop_sc(src2d, dst2d, hs):
    @functools.partial(
        pl.kernel,
        out_type=jax.ShapeDtypeStruct((NC, N2, C), jnp.float32),
        mesh=_SC_MESH,
        compiler_params=_SC_PARAMS,
        scratch_types=[
            pltpu.VMEM_SHARED((N2, C), jnp.float32),
            pltpu.VMEM((CHUNK,), jnp.int32),
            pltpu.VMEM((1, CHUNK), jnp.int32),
            pltpu.VMEM((CHUNK, C), jnp.float32),
            pltpu.VMEM((ZR, C), jnp.float32),
        ],
    )
    def k(src_hbm, dst_hbm, hs_hbm, out_hbm, racc, src_v, dst_v, rows_v, zer_v):
        cid = lax.axis_index("c")
        sid = lax.axis_index("s")
        wid = sid * NC + cid

        @pl.loop(0, ZR)
        def _(r):
            @pl.loop(0, C, step=16)
            def _(cc):
                zer_v.at[pl.ds(r, 1), pl.ds(cc, 16)][...] = jnp.zeros(
                    (1, 16), jnp.float32
                )

        @pl.loop(0, RPS, step=ZR)
        def _(b):
            pltpu.sync_copy(zer_v, racc.at[pl.ds(sid * RPS + b, ZR)])

        plsc.subcore_barrier()

        def chunk(r):
            pltpu.sync_copy(src_hbm.at[r], src_v)
            pltpu.sync_copy(dst_hbm.at[r], dst_v.at[0])
            pltpu.sync_copy(hs_hbm.at[src_v], rows_v)
            pltpu.sync_copy(rows_v, racc.at[dst_v.at[0]], add=True)

        @pl.loop(0, FULL)
        def _(j):
            chunk(wid + j * NW)

        @pl.when(wid < EXTRA)
        def _():
            chunk(wid + FULL * NW)

        plsc.subcore_barrier()
        pltpu.sync_copy(
            racc.at[pl.ds(sid * RPS, RPS)],
            out_hbm.at[cid, pl.ds(sid * RPS, RPS)],
        )

    return k(src2d, dst2d, hs)


def kernel(x, edge_index, W1, b1, W2, b2):
    src2d = edge_index[0].reshape(NCHUNK, CHUNK)
    dst2d = edge_index[1].reshape(NCHUNK, CHUNK)

    h0 = _mlp(x, W1, b1, W2, b2)
    dp = _deg_sc(dst2d)
    dinv, hs = _prep(dp, h0)

    h = h0
    for _ in range(K):
        p = _prop_sc(src2d, dst2d, hs)
        h, hs = _combine(p, hs, h0, dinv)
    return h


# speedup vs baseline: 30.8559x; 2.5100x over previous
import functools

import jax
import jax.numpy as jnp
from jax import lax
from jax.experimental import pallas as pl
from jax.experimental.pallas import tpu as pltpu
from jax.experimental.pallas import tpu_sc as plsc

N = 10000
E = 320000
F_IN = 128
HID = 128
C = 64
K = 10
ALPHA = 0.1

NC = 2
NS = 16
NW = NC * NS
CHUNK = 128
CHPW = 80
EPAD = NW * CHPW * CHUNK
N2 = 10240
RPS = N2 // NS
ZR = 64
NBUF = 4
LOOK = 2

_SC_MESH = plsc.VectorSubcoreMesh(core_axis_name="c", subcore_axis_name="s")
_SC_PARAMS = pltpu.CompilerParams(use_tc_tiling_on_sc=False)


_MLP_BLK = 1000


def _mlp_body(x_ref, w1_ref, b1_ref, w2_ref, b2_ref, o_ref):
    h = jnp.dot(x_ref[...], w1_ref[...], preferred_element_type=jnp.float32)
    h = jnp.maximum(h + b1_ref[...], 0.0)
    o_ref[...] = (
        jnp.dot(h, w2_ref[...], preferred_element_type=jnp.float32) + b2_ref[...]
    )


def _mlp(x, W1, b1, W2, b2):
    grid = (N // _MLP_BLK,)
    return pl.pallas_call(
        _mlp_body,
        grid=grid,
        in_specs=[
            pl.BlockSpec((_MLP_BLK, F_IN), lambda i: (i, 0)),
            pl.BlockSpec((F_IN, HID), lambda i: (0, 0)),
            pl.BlockSpec((1, HID), lambda i: (0, 0)),
            pl.BlockSpec((HID, C), lambda i: (0, 0)),
            pl.BlockSpec((1, C), lambda i: (0, 0)),
        ],
        out_specs=pl.BlockSpec((_MLP_BLK, C), lambda i: (i, 0)),
        out_shape=jax.ShapeDtypeStruct((N, C), jnp.float32),
    )(x, W1, b1.reshape(1, HID), W2, b2.reshape(1, C))


_ROW_BLK = 1000


def _prep_body(dp_ref, h0_ref, dinv_ref, hs_ref):
    deg = dp_ref[0, :, 0:1] + dp_ref[1, :, 0:1] + 1.0
    dinv = lax.rsqrt(deg)
    dinv_ref[...] = dinv
    hs_ref[...] = dinv * h0_ref[...]


def _prep(dp, h0):
    grid = (N // _ROW_BLK,)
    return pl.pallas_call(
        _prep_body,
        grid=grid,
        in_specs=[
            pl.BlockSpec((2, _ROW_BLK, 16), lambda i: (0, i, 0)),
            pl.BlockSpec((_ROW_BLK, C), lambda i: (i, 0)),
        ],
        out_specs=[
            pl.BlockSpec((_ROW_BLK, 1), lambda i: (i, 0)),
            pl.BlockSpec((_ROW_BLK, C), lambda i: (i, 0)),
        ],
        out_shape=[
            jax.ShapeDtypeStruct((N, 1), jnp.float32),
            jax.ShapeDtypeStruct((N, C), jnp.float32),
        ],
    )(dp, h0)


def _combine_body(p_ref, hs_ref, h0_ref, dinv_ref, hn_ref, hsn_ref):
    dinv = dinv_ref[...]
    agg = dinv * (p_ref[0] + p_ref[1] + hs_ref[...])
    hn = (1.0 - ALPHA) * agg + ALPHA * h0_ref[...]
    hn_ref[...] = hn
    hsn_ref[...] = dinv * hn


def _combine(p, hs, h0, dinv):
    grid = (N // _ROW_BLK,)
    return pl.pallas_call(
        _combine_body,
        grid=grid,
        in_specs=[
            pl.BlockSpec((2, _ROW_BLK, C), lambda i: (0, i, 0)),
            pl.BlockSpec((_ROW_BLK, C), lambda i: (i, 0)),
            pl.BlockSpec((_ROW_BLK, C), lambda i: (i, 0)),
            pl.BlockSpec((_ROW_BLK, 1), lambda i: (i, 0)),
        ],
        out_specs=[
            pl.BlockSpec((_ROW_BLK, C), lambda i: (i, 0)),
            pl.BlockSpec((_ROW_BLK, C), lambda i: (i, 0)),
        ],
        out_shape=[
            jax.ShapeDtypeStruct((N, C), jnp.float32),
            jax.ShapeDtypeStruct((N, C), jnp.float32),
        ],
    )(p, hs, h0, dinv)


def _deg_sc(dst3):
    @functools.partial(
        pl.kernel,
        out_type=jax.ShapeDtypeStruct((NC, N2, 16), jnp.float32),
        mesh=_SC_MESH,
        compiler_params=_SC_PARAMS,
        scratch_types=[
            pltpu.VMEM_SHARED((N2, 16), jnp.float32),
            pltpu.VMEM((CHPW, CHUNK), jnp.int32),
            pltpu.VMEM((CHUNK, 16), jnp.float32),
            pltpu.VMEM((RPS, 16), jnp.float32),
            pltpu.SemaphoreType.DMA,
            pltpu.SemaphoreType.DMA,
        ],
    )
    def k(dst_hbm, out_hbm, dacc, dst_v, ones_v, zer_v, isem, ssem):
        cid = lax.axis_index("c")
        sid = lax.axis_index("s")
        wid = sid * NC + cid

        pltpu.async_copy(dst_hbm.at[wid], dst_v, isem)

        @pl.loop(0, CHUNK)
        def _(r):
            ones_v.at[pl.ds(r, 1), pl.ds(0, 16)][...] = jnp.ones((1, 16), jnp.float32)

        @pl.loop(0, RPS)
        def _(r):
            zer_v.at[pl.ds(r, 1), pl.ds(0, 16)][...] = jnp.zeros((1, 16), jnp.float32)

        pltpu.sync_copy(zer_v, dacc.at[pl.ds(sid * RPS, RPS)])
        pltpu.make_async_copy(dst_hbm.at[wid], dst_v, isem).wait()
        plsc.subcore_barrier()

        @pl.loop(0, CHPW)
        def _(j):
            pltpu.async_copy(ones_v, dacc.at[dst_v.at[j]], ssem, add=True)

        @pl.loop(0, CHPW)
        def _(j):
            pltpu.make_async_copy(ones_v, dacc.at[dst_v.at[j]], ssem).wait()

        plsc.subcore_barrier()
        pltpu.sync_copy(
            dacc.at[pl.ds(sid * RPS, RPS)],
            out_hbm.at[cid, pl.ds(sid * RPS, RPS)],
        )

    return k(dst3)


def _prop_sc(src3, dst3, hs):
    @functools.partial(
        pl.kernel,
        out_type=jax.ShapeDtypeStruct((NC, N2, C), jnp.float32),
        mesh=_SC_MESH,
        compiler_params=_SC_PARAMS,
        scratch_types=[
            pltpu.VMEM_SHARED((N2, C), jnp.float32),
            pltpu.VMEM((CHPW, CHUNK), jnp.int32),
            pltpu.VMEM((CHPW, CHUNK), jnp.int32),
            pltpu.VMEM((NBUF, CHUNK, C), jnp.float32),
            pltpu.VMEM((ZR, C), jnp.float32),
            pltpu.SemaphoreType.DMA,
            pltpu.SemaphoreType.DMA((NBUF,)),
            pltpu.SemaphoreType.DMA((NBUF,)),
        ],
    )
    def k(src_hbm, dst_hbm, hs_hbm, out_hbm, racc, src_v, dst_v, rows_v, zer_v,
          isem, gsem, ssem):
        cid = lax.axis_index("c")
        sid = lax.axis_index("s")
        wid = sid * NC + cid

        pltpu.async_copy(src_hbm.at[wid], src_v, isem)
        pltpu.async_copy(dst_hbm.at[wid], dst_v, isem)

        @pl.loop(0, ZR)
        def _(r):
            @pl.loop(0, C, step=16)
            def _(cc):
                zer_v.at[pl.ds(r, 1), pl.ds(cc, 16)][...] = jnp.zeros(
                    (1, 16), jnp.float32
                )

        @pl.loop(0, RPS, step=ZR)
        def _(b):
            pltpu.sync_copy(zer_v, racc.at[pl.ds(sid * RPS + b, ZR)])

        pltpu.make_async_copy(src_hbm.at[wid], src_v, isem).wait()
        pltpu.make_async_copy(dst_hbm.at[wid], dst_v, isem).wait()

        def g_fire(j, slot):
            pltpu.async_copy(hs_hbm.at[src_v.at[j]], rows_v.at[slot], gsem.at[slot])

        def g_wait(j, slot):
            pltpu.make_async_copy(
                hs_hbm.at[src_v.at[j]], rows_v.at[slot], gsem.at[slot]
            ).wait()

        def s_fire(j, slot):
            pltpu.async_copy(
                rows_v.at[slot], racc.at[dst_v.at[j]], ssem.at[slot], add=True
            )

        def s_wait(j, slot):
            pltpu.make_async_copy(
                rows_v.at[slot], racc.at[dst_v.at[j]], ssem.at[slot]
            ).wait()

        for b in range(LOOK):
            g_fire(b, b)

        plsc.subcore_barrier()

        for b in range(NBUF):
            if b + LOOK < NBUF:
                g_fire(b + LOOK, b + LOOK)
            else:
                s_wait(b + LOOK - NBUF, (b + LOOK) % NBUF)
                g_fire(b + LOOK, (b + LOOK) % NBUF)
            g_wait(b, b)
            s_fire(b, b)

        @pl.loop(1, CHPW // NBUF - 1)
        def _(g):
            j0 = g * NBUF
            for b in range(NBUF):
                sl = (b + LOOK) % NBUF
                s_wait(j0 + b + LOOK - NBUF, sl)
                g_fire(j0 + b + LOOK, sl)
                g_wait(j0 + b, b)
                s_fire(j0 + b, b)

        j0 = CHPW - NBUF
        for b in range(NBUF):
            sl = (b + LOOK) % NBUF
            s_wait(j0 + b + LOOK - NBUF, sl)
            if b < NBUF - LOOK:
                g_fire(j0 + b + LOOK, sl)
            g_wait(j0 + b, b)
            s_fire(j0 + b, b)

        for b in range(LOOK):
            sl = (CHPW - LOOK + b) % NBUF
            s_wait(CHPW - LOOK + b, sl)

        plsc.subcore_barrier()
        pltpu.sync_copy(
            racc.at[pl.ds(sid * RPS, RPS)],
            out_hbm.at[cid, pl.ds(sid * RPS, RPS)],
        )

    return k(src3, dst3, hs)


def kernel(x, edge_index, W1, b1, W2, b2):
    npad = EPAD - E
    pad_src = (jnp.arange(npad, dtype=jnp.int32) * 97) % N
    pad_dst = N + (jnp.arange(npad, dtype=jnp.int32) % (N2 - N))
    src3 = jnp.concatenate([edge_index[0], pad_src]).reshape(NW, CHPW, CHUNK)
    dst3 = jnp.concatenate([edge_index[1], pad_dst]).reshape(NW, CHPW, CHUNK)

    h0 = _mlp(x, W1, b1, W2, b2)
    dp = _deg_sc(dst3)
    dinv, hs = _prep(dp, h0)

    h = h0
    for _ in range(K):
        p = _prop_sc(src3, dst3, hs)
        h, hs = _combine(p, hs, h0, dinv)
    return h


# speedup vs baseline: 31.5907x; 1.0238x over previous
import functools

import jax
import jax.numpy as jnp
from jax import lax
from jax.experimental import pallas as pl
from jax.experimental.pallas import tpu as pltpu
from jax.experimental.pallas import tpu_sc as plsc

N = 10000
E = 320000
F_IN = 128
HID = 128
C = 64
K = 10
ALPHA = 0.1

NC = 2
NS = 16
NW = NC * NS
CHUNK = 128
CHPW = 80
EPAD = NW * CHPW * CHUNK
N2 = 10240
RPS = N2 // NS
ZR = 32
NBUF = 8
LOOK = 4

_SC_MESH = plsc.VectorSubcoreMesh(core_axis_name="c", subcore_axis_name="s")
_SC_PARAMS = pltpu.CompilerParams(use_tc_tiling_on_sc=False)


_MLP_BLK = 1000


def _mlp_body(x_ref, w1_ref, b1_ref, w2_ref, b2_ref, o_ref):
    h = jnp.dot(x_ref[...], w1_ref[...], preferred_element_type=jnp.float32)
    h = jnp.maximum(h + b1_ref[...], 0.0)
    o_ref[...] = (
        jnp.dot(h, w2_ref[...], preferred_element_type=jnp.float32) + b2_ref[...]
    )


def _mlp(x, W1, b1, W2, b2):
    grid = (N // _MLP_BLK,)
    return pl.pallas_call(
        _mlp_body,
        grid=grid,
        in_specs=[
            pl.BlockSpec((_MLP_BLK, F_IN), lambda i: (i, 0)),
            pl.BlockSpec((F_IN, HID), lambda i: (0, 0)),
            pl.BlockSpec((1, HID), lambda i: (0, 0)),
            pl.BlockSpec((HID, C), lambda i: (0, 0)),
            pl.BlockSpec((1, C), lambda i: (0, 0)),
        ],
        out_specs=pl.BlockSpec((_MLP_BLK, C), lambda i: (i, 0)),
        out_shape=jax.ShapeDtypeStruct((N, C), jnp.float32),
    )(x, W1, b1.reshape(1, HID), W2, b2.reshape(1, C))


_ROW_BLK = 1000


def _prep_body(dp_ref, h0_ref, dinv_ref, hs_ref):
    deg = dp_ref[0, :, 0:1] + dp_ref[1, :, 0:1] + 1.0
    dinv = lax.rsqrt(deg)
    dinv_ref[...] = dinv
    hs_ref[...] = dinv * h0_ref[...]


def _prep(dp, h0):
    grid = (N // _ROW_BLK,)
    return pl.pallas_call(
        _prep_body,
        grid=grid,
        in_specs=[
            pl.BlockSpec((2, _ROW_BLK, 16), lambda i: (0, i, 0)),
            pl.BlockSpec((_ROW_BLK, C), lambda i: (i, 0)),
        ],
        out_specs=[
            pl.BlockSpec((_ROW_BLK, 1), lambda i: (i, 0)),
            pl.BlockSpec((_ROW_BLK, C), lambda i: (i, 0)),
        ],
        out_shape=[
            jax.ShapeDtypeStruct((N, 1), jnp.float32),
            jax.ShapeDtypeStruct((N, C), jnp.float32),
        ],
    )(dp, h0)


def _combine_body(p_ref, hs_ref, h0_ref, dinv_ref, hn_ref, hsn_ref):
    dinv = dinv_ref[...]
    agg = dinv * (p_ref[0] + p_ref[1] + hs_ref[...])
    hn = (1.0 - ALPHA) * agg + ALPHA * h0_ref[...]
    hn_ref[...] = hn
    hsn_ref[...] = dinv * hn


def _combine(p, hs, h0, dinv):
    grid = (N // _ROW_BLK,)
    return pl.pallas_call(
        _combine_body,
        grid=grid,
        in_specs=[
            pl.BlockSpec((2, _ROW_BLK, C), lambda i: (0, i, 0)),
            pl.BlockSpec((_ROW_BLK, C), lambda i: (i, 0)),
            pl.BlockSpec((_ROW_BLK, C), lambda i: (i, 0)),
            pl.BlockSpec((_ROW_BLK, 1), lambda i: (i, 0)),
        ],
        out_specs=[
            pl.BlockSpec((_ROW_BLK, C), lambda i: (i, 0)),
            pl.BlockSpec((_ROW_BLK, C), lambda i: (i, 0)),
        ],
        out_shape=[
            jax.ShapeDtypeStruct((N, C), jnp.float32),
            jax.ShapeDtypeStruct((N, C), jnp.float32),
        ],
    )(p, hs, h0, dinv)


def _deg_sc(dst3):
    @functools.partial(
        pl.kernel,
        out_type=jax.ShapeDtypeStruct((NC, N2, 16), jnp.float32),
        mesh=_SC_MESH,
        compiler_params=_SC_PARAMS,
        scratch_types=[
            pltpu.VMEM_SHARED((N2, 16), jnp.float32),
            pltpu.VMEM((CHPW, CHUNK), jnp.int32),
            pltpu.VMEM((CHUNK, 16), jnp.float32),
            pltpu.VMEM((RPS, 16), jnp.float32),
            pltpu.SemaphoreType.DMA,
            pltpu.SemaphoreType.DMA,
        ],
    )
    def k(dst_hbm, out_hbm, dacc, dst_v, ones_v, zer_v, isem, ssem):
        cid = lax.axis_index("c")
        sid = lax.axis_index("s")
        wid = sid * NC + cid

        pltpu.async_copy(dst_hbm.at[wid], dst_v, isem)

        @pl.loop(0, CHUNK)
        def _(r):
            ones_v.at[pl.ds(r, 1), pl.ds(0, 16)][...] = jnp.ones((1, 16), jnp.float32)

        @pl.loop(0, RPS)
        def _(r):
            zer_v.at[pl.ds(r, 1), pl.ds(0, 16)][...] = jnp.zeros((1, 16), jnp.float32)

        pltpu.sync_copy(zer_v, dacc.at[pl.ds(sid * RPS, RPS)])
        pltpu.make_async_copy(dst_hbm.at[wid], dst_v, isem).wait()
        plsc.subcore_barrier()

        @pl.loop(0, CHPW)
        def _(j):
            pltpu.async_copy(ones_v, dacc.at[dst_v.at[j]], ssem, add=True)

        @pl.loop(0, CHPW)
        def _(j):
            pltpu.make_async_copy(ones_v, dacc.at[dst_v.at[j]], ssem).wait()

        plsc.subcore_barrier()
        pltpu.sync_copy(
            dacc.at[pl.ds(sid * RPS, RPS)],
            out_hbm.at[cid, pl.ds(sid * RPS, RPS)],
        )

    return k(dst3)


def _prop_sc(src3, dst3, hs):
    @functools.partial(
        pl.kernel,
        out_type=jax.ShapeDtypeStruct((NC, N2, C), jnp.float32),
        mesh=_SC_MESH,
        compiler_params=_SC_PARAMS,
        scratch_types=[
            pltpu.VMEM_SHARED((N2, C), jnp.float32),
            pltpu.VMEM((CHPW, CHUNK), jnp.int32),
            pltpu.VMEM((CHPW, CHUNK), jnp.int32),
            pltpu.VMEM((NBUF, CHUNK, C), jnp.float32),
            pltpu.VMEM((ZR, C), jnp.float32),
            pltpu.SemaphoreType.DMA,
            pltpu.SemaphoreType.DMA((NBUF,)),
            pltpu.SemaphoreType.DMA((NBUF,)),
        ],
    )
    def k(src_hbm, dst_hbm, hs_hbm, out_hbm, racc, src_v, dst_v, rows_v, zer_v,
          isem, gsem, ssem):
        cid = lax.axis_index("c")
        sid = lax.axis_index("s")
        wid = sid * NC + cid

        pltpu.async_copy(src_hbm.at[wid], src_v, isem)
        pltpu.async_copy(dst_hbm.at[wid], dst_v, isem)

        @pl.loop(0, ZR)
        def _(r):
            @pl.loop(0, C, step=16)
            def _(cc):
                zer_v.at[pl.ds(r, 1), pl.ds(cc, 16)][...] = jnp.zeros(
                    (1, 16), jnp.float32
                )

        @pl.loop(0, RPS, step=ZR)
        def _(b):
            pltpu.sync_copy(zer_v, racc.at[pl.ds(sid * RPS + b, ZR)])

        pltpu.make_async_copy(src_hbm.at[wid], src_v, isem).wait()
        pltpu.make_async_copy(dst_hbm.at[wid], dst_v, isem).wait()

        def g_fire(j, slot):
            pltpu.async_copy(hs_hbm.at[src_v.at[j]], rows_v.at[slot], gsem.at[slot])

        def g_wait(j, slot):
            pltpu.make_async_copy(
                hs_hbm.at[src_v.at[j]], rows_v.at[slot], gsem.at[slot]
            ).wait()

        def s_fire(j, slot):
            pltpu.async_copy(
                rows_v.at[slot], racc.at[dst_v.at[j]], ssem.at[slot], add=True
            )

        def s_wait(j, slot):
            pltpu.make_async_copy(
                rows_v.at[slot], racc.at[dst_v.at[j]], ssem.at[slot]
            ).wait()

        for b in range(LOOK):
            g_fire(b, b)

        plsc.subcore_barrier()

        for b in range(NBUF):
            if b + LOOK < NBUF:
                g_fire(b + LOOK, b + LOOK)
            else:
                s_wait(b + LOOK - NBUF, (b + LOOK) % NBUF)
                g_fire(b + LOOK, (b + LOOK) % NBUF)
            g_wait(b, b)
            s_fire(b, b)

        @pl.loop(1, CHPW // NBUF - 1)
        def _(g):
            j0 = g * NBUF
            for b in range(NBUF):
                sl = (b + LOOK) % NBUF
                s_wait(j0 + b + LOOK - NBUF, sl)
                g_fire(j0 + b + LOOK, sl)
                g_wait(j0 + b, b)
                s_fire(j0 + b, b)

        j0 = CHPW - NBUF
        for b in range(NBUF):
            sl = (b + LOOK) % NBUF
            s_wait(j0 + b + LOOK - NBUF, sl)
            if b < NBUF - LOOK:
                g_fire(j0 + b + LOOK, sl)
            g_wait(j0 + b, b)
            s_fire(j0 + b, b)

        for b in range(LOOK):
            sl = (CHPW - LOOK + b) % NBUF
            s_wait(CHPW - LOOK + b, sl)

        plsc.subcore_barrier()
        pltpu.sync_copy(
            racc.at[pl.ds(sid * RPS, RPS)],
            out_hbm.at[cid, pl.ds(sid * RPS, RPS)],
        )

    return k(src3, dst3, hs)


def kernel(x, edge_index, W1, b1, W2, b2):
    npad = EPAD - E
    pad_src = (jnp.arange(npad, dtype=jnp.int32) * 97) % N
    pad_dst = N + (jnp.arange(npad, dtype=jnp.int32) % (N2 - N))
    src3 = jnp.concatenate([edge_index[0], pad_src]).reshape(NW, CHPW, CHUNK)
    dst3 = jnp.concatenate([edge_index[1], pad_dst]).reshape(NW, CHPW, CHUNK)

    h0 = _mlp(x, W1, b1, W2, b2)
    dp = _deg_sc(dst3)
    dinv, hs = _prep(dp, h0)

    h = h0
    for _ in range(K):
        p = _prop_sc(src3, dst3, hs)
        h, hs = _combine(p, hs, h0, dinv)
    return h


# speedup vs baseline: 39.5687x; 1.2525x over previous
import functools

import jax
import jax.numpy as jnp
from jax import lax
from jax.experimental import pallas as pl
from jax.experimental.pallas import tpu as pltpu
from jax.experimental.pallas import tpu_sc as plsc

N = 10000
E = 320000
F_IN = 128
HID = 128
C = 64
K = 10
ALPHA = 0.1

NC = 2
NS = 16
NW = NC * NS
CHUNK = 128
CHPW = 80
EPAD = NW * CHPW * CHUNK
N2 = 10240
RPS = N2 // NS
ZR = 32
NBUF = 8
LOOK = 4

_SC_MESH = plsc.VectorSubcoreMesh(core_axis_name="c", subcore_axis_name="s")
_SC_PARAMS = pltpu.CompilerParams(use_tc_tiling_on_sc=False)


_MLP_BLK = 1024


def _mlp_body(x_ref, w1_ref, b1_ref, w2_ref, b2_ref, o_ref):
    h = jnp.dot(x_ref[...], w1_ref[...], preferred_element_type=jnp.float32)
    h = jnp.maximum(h + b1_ref[...], 0.0)
    o_ref[...] = (
        jnp.dot(h, w2_ref[...], preferred_element_type=jnp.float32) + b2_ref[...]
    )


def _mlp(x2, W1, b1, W2, b2):
    grid = (N2 // _MLP_BLK,)
    return pl.pallas_call(
        _mlp_body,
        grid=grid,
        in_specs=[
            pl.BlockSpec((_MLP_BLK, F_IN), lambda i: (i, 0)),
            pl.BlockSpec((F_IN, HID), lambda i: (0, 0)),
            pl.BlockSpec((1, HID), lambda i: (0, 0)),
            pl.BlockSpec((HID, C), lambda i: (0, 0)),
            pl.BlockSpec((1, C), lambda i: (0, 0)),
        ],
        out_specs=pl.BlockSpec((_MLP_BLK, C), lambda i: (i, 0)),
        out_shape=jax.ShapeDtypeStruct((N2, C), jnp.float32),
    )(x2, W1, b1.reshape(1, HID), W2, b2.reshape(1, C))


_ROW_BLK = 1024


def _prep_body(dp_ref, h0_ref, hs_ref, d2_ref, g0_ref, sq_ref):
    deg = dp_ref[0, :, 0:1] + dp_ref[1, :, 0:1] + 1.0
    dinv = lax.rsqrt(deg)
    h0 = h0_ref[...]
    hs_ref[...] = dinv * h0
    d2_ref[...] = jnp.broadcast_to((1.0 - ALPHA) / deg, h0.shape)
    g0_ref[...] = (ALPHA * dinv) * h0
    sq_ref[...] = jnp.broadcast_to(jnp.sqrt(deg), h0.shape)


def _prep(dp, h0p):
    grid = (N2 // _ROW_BLK,)
    specs = pl.BlockSpec((_ROW_BLK, C), lambda i: (i, 0))
    shape = jax.ShapeDtypeStruct((N2, C), jnp.float32)
    return pl.pallas_call(
        _prep_body,
        grid=grid,
        in_specs=[
            pl.BlockSpec((2, _ROW_BLK, 16), lambda i: (0, i, 0)),
            specs,
        ],
        out_specs=[specs, specs, specs, specs],
        out_shape=[shape, shape, shape, shape],
    )(dp, h0p)


_FLAT = N2 * C
_FBLK = 65536
_FGRID = _FLAT // _FBLK


def _combine_body(p_ref, q_ref, hs_ref, d2_ref, g0_ref, o_ref):
    o_ref[...] = (
        d2_ref[...] * (p_ref[...] + q_ref[...] + hs_ref[...]) + g0_ref[...]
    )


def _combine(p1, hs1, d2b1, g0b1):
    spec = pl.BlockSpec((_FBLK,), lambda i: (i,))
    return pl.pallas_call(
        _combine_body,
        grid=(_FGRID,),
        in_specs=[
            pl.BlockSpec((_FBLK,), lambda i: (i,)),
            pl.BlockSpec((_FBLK,), lambda i: (i + _FGRID,)),
            spec,
            spec,
            spec,
        ],
        out_specs=spec,
        out_shape=jax.ShapeDtypeStruct((_FLAT,), jnp.float32),
    )(p1, p1, hs1, d2b1, g0b1)


def _finish_body(hs_ref, sq_ref, o_ref):
    o_ref[...] = hs_ref[...] * sq_ref[...]


def _finish(hs1, sqb1):
    spec = pl.BlockSpec((_FBLK,), lambda i: (i,))
    return pl.pallas_call(
        _finish_body,
        grid=(_FGRID,),
        in_specs=[spec, spec],
        out_specs=spec,
        out_shape=jax.ShapeDtypeStruct((_FLAT,), jnp.float32),
    )(hs1, sqb1)


def _deg_sc(dst3):
    @functools.partial(
        pl.kernel,
        out_type=jax.ShapeDtypeStruct((NC, N2, 16), jnp.float32),
        mesh=_SC_MESH,
        compiler_params=_SC_PARAMS,
        scratch_types=[
            pltpu.VMEM_SHARED((N2, 16), jnp.float32),
            pltpu.VMEM((CHPW, CHUNK), jnp.int32),
            pltpu.VMEM((CHUNK, 16), jnp.float32),
            pltpu.VMEM((RPS, 16), jnp.float32),
            pltpu.SemaphoreType.DMA,
            pltpu.SemaphoreType.DMA,
        ],
    )
    def k(dst_hbm, out_hbm, dacc, dst_v, ones_v, zer_v, isem, ssem):
        cid = lax.axis_index("c")
        sid = lax.axis_index("s")
        wid = sid * NC + cid

        pltpu.async_copy(dst_hbm.at[wid], dst_v, isem)

        @pl.loop(0, CHUNK)
        def _(r):
            ones_v.at[pl.ds(r, 1), pl.ds(0, 16)][...] = jnp.ones((1, 16), jnp.float32)

        @pl.loop(0, RPS)
        def _(r):
            zer_v.at[pl.ds(r, 1), pl.ds(0, 16)][...] = jnp.zeros((1, 16), jnp.float32)

        pltpu.sync_copy(zer_v, dacc.at[pl.ds(sid * RPS, RPS)])
        pltpu.make_async_copy(dst_hbm.at[wid], dst_v, isem).wait()
        plsc.subcore_barrier()

        @pl.loop(0, CHPW)
        def _(j):
            pltpu.async_copy(ones_v, dacc.at[dst_v.at[j]], ssem, add=True)

        @pl.loop(0, CHPW)
        def _(j):
            pltpu.make_async_copy(ones_v, dacc.at[dst_v.at[j]], ssem).wait()

        plsc.subcore_barrier()
        pltpu.sync_copy(
            dacc.at[pl.ds(sid * RPS, RPS)],
            out_hbm.at[cid, pl.ds(sid * RPS, RPS)],
        )

    return k(dst3)


def _prop_sc(src3, dst3, hs2):
    @functools.partial(
        pl.kernel,
        out_type=jax.ShapeDtypeStruct((NC, N2, C), jnp.float32),
        mesh=_SC_MESH,
        compiler_params=_SC_PARAMS,
        scratch_types=[
            pltpu.VMEM_SHARED((N2, C), jnp.float32),
            pltpu.VMEM((CHPW, CHUNK), jnp.int32),
            pltpu.VMEM((CHPW, CHUNK), jnp.int32),
            pltpu.VMEM((NBUF, CHUNK, C), jnp.float32),
            pltpu.VMEM((ZR, C), jnp.float32),
            pltpu.SemaphoreType.DMA,
            pltpu.SemaphoreType.DMA,
            pltpu.SemaphoreType.DMA((NBUF,)),
            pltpu.SemaphoreType.DMA((NBUF,)),
        ],
    )
    def k(src_hbm, dst_hbm, hs_hbm, out_hbm, racc, src_v, dst_v, rows_v, zer_v,
          isem, zsem, gsem, ssem):
        cid = lax.axis_index("c")
        sid = lax.axis_index("s")
        wid = sid * NC + cid

        pltpu.async_copy(src_hbm.at[wid], src_v, isem)
        pltpu.async_copy(dst_hbm.at[wid], dst_v, isem)

        @pl.loop(0, ZR)
        def _(r):
            @pl.loop(0, C, step=16)
            def _(cc):
                zer_v.at[pl.ds(r, 1), pl.ds(cc, 16)][...] = jnp.zeros(
                    (1, 16), jnp.float32
                )

        @pl.loop(0, RPS, step=ZR)
        def _(b):
            pltpu.async_copy(zer_v, racc.at[pl.ds(sid * RPS + b, ZR)], zsem)

        pltpu.make_async_copy(src_hbm.at[wid], src_v, isem).wait()
        pltpu.make_async_copy(dst_hbm.at[wid], dst_v, isem).wait()

        def g_fire(j, slot):
            pltpu.async_copy(hs_hbm.at[src_v.at[j]], rows_v.at[slot], gsem.at[slot])

        def g_wait(j, slot):
            pltpu.make_async_copy(
                hs_hbm.at[src_v.at[j]], rows_v.at[slot], gsem.at[slot]
            ).wait()

        def s_fire(j, slot):
            pltpu.async_copy(
                rows_v.at[slot], racc.at[dst_v.at[j]], ssem.at[slot], add=True
            )

        def s_wait(j, slot):
            pltpu.make_async_copy(
                rows_v.at[slot], racc.at[dst_v.at[j]], ssem.at[slot]
            ).wait()

        for b in range(LOOK):
            g_fire(b, b)

        @pl.loop(0, RPS, step=ZR)
        def _(b):
            pltpu.make_async_copy(zer_v, racc.at[pl.ds(sid * RPS + b, ZR)], zsem).wait()

        plsc.subcore_barrier()

        for b in range(NBUF):
            if b + LOOK < NBUF:
                g_fire(b + LOOK, b + LOOK)
            else:
                s_wait(b + LOOK - NBUF, (b + LOOK) % NBUF)
                g_fire(b + LOOK, (b + LOOK) % NBUF)
            g_wait(b, b)
            s_fire(b, b)

        @pl.loop(1, CHPW // NBUF - 1)
        def _(g):
            j0 = g * NBUF
            for b in range(NBUF):
                sl = (b + LOOK) % NBUF
                s_wait(j0 + b + LOOK - NBUF, sl)
                g_fire(j0 + b + LOOK, sl)
                g_wait(j0 + b, b)
                s_fire(j0 + b, b)

        j0 = CHPW - NBUF
        for b in range(NBUF):
            sl = (b + LOOK) % NBUF
            s_wait(j0 + b + LOOK - NBUF, sl)
            if b < NBUF - LOOK:
                g_fire(j0 + b + LOOK, sl)
            g_wait(j0 + b, b)
            s_fire(j0 + b, b)

        for b in range(LOOK):
            sl = (CHPW - LOOK + b) % NBUF
            s_wait(CHPW - LOOK + b, sl)

        plsc.subcore_barrier()
        pltpu.sync_copy(
            racc.at[pl.ds(sid * RPS, RPS)],
            out_hbm.at[cid, pl.ds(sid * RPS, RPS)],
        )

    return k(src3, dst3, hs2)


def kernel(x, edge_index, W1, b1, W2, b2):
    npad = EPAD - E
    pad_src = (jnp.arange(npad, dtype=jnp.int32) * 97) % N
    pad_dst = N + (jnp.arange(npad, dtype=jnp.int32) % (N2 - N))
    src3 = jnp.concatenate([edge_index[0], pad_src]).reshape(NW, CHPW, CHUNK)
    dst3 = jnp.concatenate([edge_index[1], pad_dst]).reshape(NW, CHPW, CHUNK)

    x2 = jnp.pad(x, ((0, N2 - N), (0, 0)))
    h0p = _mlp(x2, W1, b1, W2, b2)
    dp = _deg_sc(dst3)
    hs0, d2b, g0b, sqb = _prep(dp, h0p)

    d2b1 = d2b.reshape(_FLAT)
    g0b1 = g0b.reshape(_FLAT)
    hs1 = hs0.reshape(_FLAT)
    for _ in range(K):
        p = _prop_sc(src3, dst3, hs1.reshape(N2, C))
        hs1 = _combine(p.reshape(2 * _FLAT), hs1, d2b1, g0b1)
    h1 = _finish(hs1, sqb.reshape(_FLAT))
    return h1.reshape(N2, C)[:N]


# speedup vs baseline: 39.9807x; 1.0104x over previous
import functools

import jax
import jax.numpy as jnp
from jax import lax
from jax.experimental import pallas as pl
from jax.experimental.pallas import tpu as pltpu
from jax.experimental.pallas import tpu_sc as plsc

N = 10000
E = 320000
F_IN = 128
HID = 128
C = 64
K = 10
ALPHA = 0.1

NC = 2
NS = 16
NW = NC * NS
CHUNK = 128
CHPW = 80
EPAD = NW * CHPW * CHUNK
N2 = 10240
RPS = N2 // NS
ZR = 32
NBUF = 8
LOOK = 4

_SC_MESH = plsc.VectorSubcoreMesh(core_axis_name="c", subcore_axis_name="s")
_SC_PARAMS = pltpu.CompilerParams(use_tc_tiling_on_sc=False)


_MLP_BLK = 1024


def _mlp_body(x_ref, w1_ref, b1_ref, w2_ref, b2_ref, o_ref):
    h = jnp.dot(x_ref[...], w1_ref[...], preferred_element_type=jnp.float32)
    h = jnp.maximum(h + b1_ref[...], 0.0)
    o_ref[...] = (
        jnp.dot(h, w2_ref[...], preferred_element_type=jnp.float32) + b2_ref[...]
    )


def _mlp(x2, W1, b1, W2, b2):
    grid = (N2 // _MLP_BLK,)
    return pl.pallas_call(
        _mlp_body,
        grid=grid,
        in_specs=[
            pl.BlockSpec((_MLP_BLK, F_IN), lambda i: (i, 0)),
            pl.BlockSpec((F_IN, HID), lambda i: (0, 0)),
            pl.BlockSpec((1, HID), lambda i: (0, 0)),
            pl.BlockSpec((HID, C), lambda i: (0, 0)),
            pl.BlockSpec((1, C), lambda i: (0, 0)),
        ],
        out_specs=pl.BlockSpec((_MLP_BLK, C), lambda i: (i, 0)),
        out_shape=jax.ShapeDtypeStruct((N2, C), jnp.float32),
        compiler_params=pltpu.CompilerParams(dimension_semantics=("parallel",)),
    )(x2, W1, b1.reshape(1, HID), W2, b2.reshape(1, C))


_ROW_BLK = 1024


def _prep_body(dp_ref, h0_ref, hs_ref, d2_ref, g0_ref, sq_ref):
    deg = dp_ref[0, :, 0:1] + dp_ref[1, :, 0:1] + 1.0
    dinv = lax.rsqrt(deg)
    h0 = h0_ref[...]
    hs_ref[...] = dinv * h0
    d2_ref[...] = jnp.broadcast_to((1.0 - ALPHA) / deg, h0.shape)
    g0_ref[...] = (ALPHA * dinv) * h0
    sq_ref[...] = jnp.broadcast_to(jnp.sqrt(deg), h0.shape)


def _prep(dp, h0p):
    grid = (N2 // _ROW_BLK,)
    specs = pl.BlockSpec((_ROW_BLK, C), lambda i: (i, 0))
    shape = jax.ShapeDtypeStruct((N2, C), jnp.float32)
    return pl.pallas_call(
        _prep_body,
        grid=grid,
        in_specs=[
            pl.BlockSpec((2, _ROW_BLK, 16), lambda i: (0, i, 0)),
            specs,
        ],
        out_specs=[specs, specs, specs, specs],
        out_shape=[shape, shape, shape, shape],
        compiler_params=pltpu.CompilerParams(dimension_semantics=("parallel",)),
    )(dp, h0p)


_FLAT = N2 * C
_FBLK = 65536
_FGRID = _FLAT // _FBLK


_PAR = pltpu.CompilerParams(dimension_semantics=("parallel",))


def _combine_body(p_ref, q_ref, hs_ref, d2_ref, g0_ref, o_ref):
    o_ref[...] = (
        d2_ref[...] * (p_ref[...] + q_ref[...] + hs_ref[...]) + g0_ref[...]
    )


def _last_body(p_ref, q_ref, hs_ref, d2_ref, g0_ref, sq_ref, o_ref):
    hsn = d2_ref[...] * (p_ref[...] + q_ref[...] + hs_ref[...]) + g0_ref[...]
    o_ref[...] = hsn * sq_ref[...]


def _combine(p1, hs1, d2b1, g0b1, sqb1=None):
    spec = pl.BlockSpec((_FBLK,), lambda i: (i,))
    in_specs = [
        pl.BlockSpec((_FBLK,), lambda i: (i,)),
        pl.BlockSpec((_FBLK,), lambda i: (i + _FGRID,)),
        spec,
        spec,
        spec,
    ]
    args = [p1, p1, hs1, d2b1, g0b1]
    body = _combine_body
    if sqb1 is not None:
        in_specs.append(spec)
        args.append(sqb1)
        body = _last_body
    return pl.pallas_call(
        body,
        grid=(_FGRID,),
        in_specs=in_specs,
        out_specs=spec,
        out_shape=jax.ShapeDtypeStruct((_FLAT,), jnp.float32),
        compiler_params=_PAR,
    )(*args)


def _deg_sc(dst3):
    @functools.partial(
        pl.kernel,
        out_type=jax.ShapeDtypeStruct((NC, N2, 16), jnp.float32),
        mesh=_SC_MESH,
        compiler_params=_SC_PARAMS,
        scratch_types=[
            pltpu.VMEM_SHARED((N2, 16), jnp.float32),
            pltpu.VMEM((CHPW, CHUNK), jnp.int32),
            pltpu.VMEM((CHUNK, 16), jnp.float32),
            pltpu.VMEM((RPS, 16), jnp.float32),
            pltpu.SemaphoreType.DMA,
            pltpu.SemaphoreType.DMA,
        ],
    )
    def k(dst_hbm, out_hbm, dacc, dst_v, ones_v, zer_v, isem, ssem):
        cid = lax.axis_index("c")
        sid = lax.axis_index("s")
        wid = sid * NC + cid

        pltpu.async_copy(dst_hbm.at[wid], dst_v, isem)

        @pl.loop(0, CHUNK)
        def _(r):
            ones_v.at[pl.ds(r, 1), pl.ds(0, 16)][...] = jnp.ones((1, 16), jnp.float32)

        @pl.loop(0, RPS)
        def _(r):
            zer_v.at[pl.ds(r, 1), pl.ds(0, 16)][...] = jnp.zeros((1, 16), jnp.float32)

        pltpu.sync_copy(zer_v, dacc.at[pl.ds(sid * RPS, RPS)])
        pltpu.make_async_copy(dst_hbm.at[wid], dst_v, isem).wait()
        plsc.subcore_barrier()

        @pl.loop(0, CHPW)
        def _(j):
            pltpu.async_copy(ones_v, dacc.at[dst_v.at[j]], ssem, add=True)

        @pl.loop(0, CHPW)
        def _(j):
            pltpu.make_async_copy(ones_v, dacc.at[dst_v.at[j]], ssem).wait()

        plsc.subcore_barrier()
        pltpu.sync_copy(
            dacc.at[pl.ds(sid * RPS, RPS)],
            out_hbm.at[cid, pl.ds(sid * RPS, RPS)],
        )

    return k(dst3)


def _prop_sc(src3, dst3, hs2):
    @functools.partial(
        pl.kernel,
        out_type=jax.ShapeDtypeStruct((NC, N2, C), jnp.float32),
        mesh=_SC_MESH,
        compiler_params=_SC_PARAMS,
        scratch_types=[
            pltpu.VMEM_SHARED((N2, C), jnp.float32),
            pltpu.VMEM((CHPW, CHUNK), jnp.int32),
            pltpu.VMEM((CHPW, CHUNK), jnp.int32),
            pltpu.VMEM((NBUF, CHUNK, C), jnp.float32),
            pltpu.VMEM((ZR, C), jnp.float32),
            pltpu.SemaphoreType.DMA,
            pltpu.SemaphoreType.DMA,
            pltpu.SemaphoreType.DMA((NBUF,)),
            pltpu.SemaphoreType.DMA((NBUF,)),
        ],
    )
    def k(src_hbm, dst_hbm, hs_hbm, out_hbm, racc, src_v, dst_v, rows_v, zer_v,
          isem, zsem, gsem, ssem):
        cid = lax.axis_index("c")
        sid = lax.axis_index("s")
        wid = sid * NC + cid

        pltpu.async_copy(src_hbm.at[wid], src_v, isem)
        pltpu.async_copy(dst_hbm.at[wid], dst_v, isem)

        @pl.loop(0, ZR)
        def _(r):
            @pl.loop(0, C, step=16)
            def _(cc):
                zer_v.at[pl.ds(r, 1), pl.ds(cc, 16)][...] = jnp.zeros(
                    (1, 16), jnp.float32
                )

        @pl.loop(0, RPS, step=ZR)
        def _(b):
            pltpu.async_copy(zer_v, racc.at[pl.ds(sid * RPS + b, ZR)], zsem)

        pltpu.make_async_copy(src_hbm.at[wid], src_v, isem).wait()
        pltpu.make_async_copy(dst_hbm.at[wid], dst_v, isem).wait()

        def g_fire(j, slot):
            pltpu.async_copy(hs_hbm.at[src_v.at[j]], rows_v.at[slot], gsem.at[slot])

        def g_wait(j, slot):
            pltpu.make_async_copy(
                hs_hbm.at[src_v.at[j]], rows_v.at[slot], gsem.at[slot]
            ).wait()

        def s_fire(j, slot):
            pltpu.async_copy(
                rows_v.at[slot], racc.at[dst_v.at[j]], ssem.at[slot], add=True
            )

        def s_wait(j, slot):
            pltpu.make_async_copy(
                rows_v.at[slot], racc.at[dst_v.at[j]], ssem.at[slot]
            ).wait()

        for b in range(LOOK):
            g_fire(b, b)

        @pl.loop(0, RPS, step=ZR)
        def _(b):
            pltpu.make_async_copy(zer_v, racc.at[pl.ds(sid * RPS + b, ZR)], zsem).wait()

        plsc.subcore_barrier()

        for b in range(NBUF):
            if b + LOOK < NBUF:
                g_fire(b + LOOK, b + LOOK)
            else:
                s_wait(b + LOOK - NBUF, (b + LOOK) % NBUF)
                g_fire(b + LOOK, (b + LOOK) % NBUF)
            g_wait(b, b)
            s_fire(b, b)

        @pl.loop(1, CHPW // NBUF - 1)
        def _(g):
            j0 = g * NBUF
            for b in range(NBUF):
                sl = (b + LOOK) % NBUF
                s_wait(j0 + b + LOOK - NBUF, sl)
                g_fire(j0 + b + LOOK, sl)
                g_wait(j0 + b, b)
                s_fire(j0 + b, b)

        j0 = CHPW - NBUF
        for b in range(NBUF):
            sl = (b + LOOK) % NBUF
            s_wait(j0 + b + LOOK - NBUF, sl)
            if b < NBUF - LOOK:
                g_fire(j0 + b + LOOK, sl)
            g_wait(j0 + b, b)
            s_fire(j0 + b, b)

        for b in range(LOOK):
            sl = (CHPW - LOOK + b) % NBUF
            s_wait(CHPW - LOOK + b, sl)

        plsc.subcore_barrier()
        pltpu.sync_copy(
            racc.at[pl.ds(sid * RPS, RPS)],
            out_hbm.at[cid, pl.ds(sid * RPS, RPS)],
        )

    return k(src3, dst3, hs2)


def kernel(x, edge_index, W1, b1, W2, b2):
    npad = EPAD - E
    pad_src = (jnp.arange(npad, dtype=jnp.int32) * 97) % N
    pad_dst = N + (jnp.arange(npad, dtype=jnp.int32) % (N2 - N))
    src3 = jnp.concatenate([edge_index[0], pad_src]).reshape(NW, CHPW, CHUNK)
    dst3 = jnp.concatenate([edge_index[1], pad_dst]).reshape(NW, CHPW, CHUNK)

    x2 = jnp.pad(x, ((0, N2 - N), (0, 0)))
    h0p = _mlp(x2, W1, b1, W2, b2)
    dp = _deg_sc(dst3)
    hs0, d2b, g0b, sqb = _prep(dp, h0p)

    d2b1 = d2b.reshape(_FLAT)
    g0b1 = g0b.reshape(_FLAT)
    hs1 = hs0.reshape(_FLAT)
    for step in range(K):
        p = _prop_sc(src3, dst3, hs1.reshape(N2, C))
        sq = sqb.reshape(_FLAT) if step == K - 1 else None
        hs1 = _combine(p.reshape(2 * _FLAT), hs1, d2b1, g0b1, sq)
    return hs1.reshape(N2, C)[:N]


# speedup vs baseline: 40.0481x; 1.0017x over previous
import functools

import jax
import jax.numpy as jnp
from jax import lax
from jax.experimental import pallas as pl
from jax.experimental.pallas import tpu as pltpu
from jax.experimental.pallas import tpu_sc as plsc

N = 10000
E = 320000
F_IN = 128
HID = 128
C = 64
K = 10
ALPHA = 0.1

NC = 2
NS = 16
NW = NC * NS
CHUNK = 128
CHPW = 80
EPAD = NW * CHPW * CHUNK
N2 = 10240
RPS = N2 // NS
ZR = 32
NBUF = 8
LOOK = 4

_SC_MESH = plsc.VectorSubcoreMesh(core_axis_name="c", subcore_axis_name="s")
_SC_PARAMS = pltpu.CompilerParams(use_tc_tiling_on_sc=False)


_MLP_BLK = 1024


def _mlp_body(x_ref, w1_ref, b1_ref, w2_ref, b2_ref, o_ref):
    h = jnp.dot(x_ref[...], w1_ref[...], preferred_element_type=jnp.float32)
    h = jnp.maximum(h + b1_ref[...], 0.0)
    o_ref[...] = (
        jnp.dot(h, w2_ref[...], preferred_element_type=jnp.float32) + b2_ref[...]
    )


def _mlp(x2, W1, b1, W2, b2):
    grid = (N2 // _MLP_BLK,)
    return pl.pallas_call(
        _mlp_body,
        grid=grid,
        in_specs=[
            pl.BlockSpec((_MLP_BLK, F_IN), lambda i: (i, 0)),
            pl.BlockSpec((F_IN, HID), lambda i: (0, 0)),
            pl.BlockSpec((1, HID), lambda i: (0, 0)),
            pl.BlockSpec((HID, C), lambda i: (0, 0)),
            pl.BlockSpec((1, C), lambda i: (0, 0)),
        ],
        out_specs=pl.BlockSpec((_MLP_BLK, C), lambda i: (i, 0)),
        out_shape=jax.ShapeDtypeStruct((N2, C), jnp.float32),
        compiler_params=pltpu.CompilerParams(dimension_semantics=("parallel",)),
    )(x2, W1, b1.reshape(1, HID), W2, b2.reshape(1, C))


_ROW_BLK = 1024


def _prep_body(dp_ref, h0_ref, hs_ref, d2_ref, sq_ref):
    deg = dp_ref[0, :, 0:1] + dp_ref[1, :, 0:1] + 1.0
    dinv = lax.rsqrt(deg)
    h0 = h0_ref[...]
    hs_ref[...] = dinv * h0
    d2_ref[...] = jnp.broadcast_to((1.0 - ALPHA) / deg, h0.shape)
    sq_ref[...] = jnp.broadcast_to(jnp.sqrt(deg), h0.shape)


def _prep(dp, h0p):
    grid = (N2 // _ROW_BLK,)
    specs = pl.BlockSpec((_ROW_BLK, C), lambda i: (i, 0))
    shape = jax.ShapeDtypeStruct((N2, C), jnp.float32)
    return pl.pallas_call(
        _prep_body,
        grid=grid,
        in_specs=[
            pl.BlockSpec((2, _ROW_BLK, 16), lambda i: (0, i, 0)),
            specs,
        ],
        out_specs=[specs, specs, specs],
        out_shape=[shape, shape, shape],
        compiler_params=pltpu.CompilerParams(dimension_semantics=("parallel",)),
    )(dp, h0p)


_FLAT = N2 * C
_FBLK = 65536
_FGRID = _FLAT // _FBLK


_PAR = pltpu.CompilerParams(dimension_semantics=("parallel",))


def _combine_body(p_ref, q_ref, hs_ref, d2_ref, hs0_ref, o_ref):
    o_ref[...] = (
        d2_ref[...] * (p_ref[...] + q_ref[...] + hs_ref[...])
        + ALPHA * hs0_ref[...]
    )


def _last_body(p_ref, q_ref, hs_ref, d2_ref, hs0_ref, sq_ref, o_ref):
    hsn = (
        d2_ref[...] * (p_ref[...] + q_ref[...] + hs_ref[...])
        + ALPHA * hs0_ref[...]
    )
    o_ref[...] = hsn * sq_ref[...]


def _combine(p1, hs1, d2b1, g0b1, sqb1=None):
    spec = pl.BlockSpec((_FBLK,), lambda i: (i,))
    in_specs = [
        pl.BlockSpec((_FBLK,), lambda i: (i,)),
        pl.BlockSpec((_FBLK,), lambda i: (i + _FGRID,)),
        spec,
        spec,
        spec,
    ]
    args = [p1, p1, hs1, d2b1, g0b1]
    body = _combine_body
    if sqb1 is not None:
        in_specs.append(spec)
        args.append(sqb1)
        body = _last_body
    return pl.pallas_call(
        body,
        grid=(_FGRID,),
        in_specs=in_specs,
        out_specs=spec,
        out_shape=jax.ShapeDtypeStruct((_FLAT,), jnp.float32),
        compiler_params=_PAR,
    )(*args)


def _deg_sc(dst3):
    @functools.partial(
        pl.kernel,
        out_type=jax.ShapeDtypeStruct((NC, N2, 16), jnp.float32),
        mesh=_SC_MESH,
        compiler_params=_SC_PARAMS,
        scratch_types=[
            pltpu.VMEM_SHARED((N2, 16), jnp.float32),
            pltpu.VMEM((CHPW, CHUNK), jnp.int32),
            pltpu.VMEM((CHUNK, 16), jnp.float32),
            pltpu.VMEM((RPS, 16), jnp.float32),
            pltpu.SemaphoreType.DMA,
            pltpu.SemaphoreType.DMA,
        ],
    )
    def k(dst_hbm, out_hbm, dacc, dst_v, ones_v, zer_v, isem, ssem):
        cid = lax.axis_index("c")
        sid = lax.axis_index("s")
        wid = sid * NC + cid

        pltpu.async_copy(dst_hbm.at[wid], dst_v, isem)

        @pl.loop(0, CHUNK)
        def _(r):
            ones_v.at[pl.ds(r, 1), pl.ds(0, 16)][...] = jnp.ones((1, 16), jnp.float32)

        @pl.loop(0, RPS)
        def _(r):
            zer_v.at[pl.ds(r, 1), pl.ds(0, 16)][...] = jnp.zeros((1, 16), jnp.float32)

        pltpu.sync_copy(zer_v, dacc.at[pl.ds(sid * RPS, RPS)])
        pltpu.make_async_copy(dst_hbm.at[wid], dst_v, isem).wait()
        plsc.subcore_barrier()

        @pl.loop(0, CHPW)
        def _(j):
            pltpu.async_copy(ones_v, dacc.at[dst_v.at[j]], ssem, add=True)

        @pl.loop(0, CHPW)
        def _(j):
            pltpu.make_async_copy(ones_v, dacc.at[dst_v.at[j]], ssem).wait()

        plsc.subcore_barrier()
        pltpu.sync_copy(
            dacc.at[pl.ds(sid * RPS, RPS)],
            out_hbm.at[cid, pl.ds(sid * RPS, RPS)],
        )

    return k(dst3)


def _prop_sc(src3, dst3, hs2):
    @functools.partial(
        pl.kernel,
        out_type=jax.ShapeDtypeStruct((NC, N2, C), jnp.float32),
        mesh=_SC_MESH,
        compiler_params=_SC_PARAMS,
        scratch_types=[
            pltpu.VMEM_SHARED((N2, C), jnp.float32),
            pltpu.VMEM((CHPW, CHUNK), jnp.int32),
            pltpu.VMEM((CHPW, CHUNK), jnp.int32),
            pltpu.VMEM((NBUF, CHUNK, C), jnp.float32),
            pltpu.VMEM((ZR, C), jnp.float32),
            pltpu.SemaphoreType.DMA,
            pltpu.SemaphoreType.DMA,
            pltpu.SemaphoreType.DMA((NBUF,)),
            pltpu.SemaphoreType.DMA((NBUF,)),
        ],
    )
    def k(src_hbm, dst_hbm, hs_hbm, out_hbm, racc, src_v, dst_v, rows_v, zer_v,
          isem, zsem, gsem, ssem):
        cid = lax.axis_index("c")
        sid = lax.axis_index("s")
        wid = sid * NC + cid

        pltpu.async_copy(src_hbm.at[wid], src_v, isem)
        pltpu.async_copy(dst_hbm.at[wid], dst_v, isem)

        @pl.loop(0, ZR)
        def _(r):
            @pl.loop(0, C, step=16)
            def _(cc):
                zer_v.at[pl.ds(r, 1), pl.ds(cc, 16)][...] = jnp.zeros(
                    (1, 16), jnp.float32
                )

        @pl.loop(0, RPS, step=ZR)
        def _(b):
            pltpu.async_copy(zer_v, racc.at[pl.ds(sid * RPS + b, ZR)], zsem)

        pltpu.make_async_copy(src_hbm.at[wid], src_v, isem).wait()
        pltpu.make_async_copy(dst_hbm.at[wid], dst_v, isem).wait()

        def g_fire(j, slot):
            pltpu.async_copy(hs_hbm.at[src_v.at[j]], rows_v.at[slot], gsem.at[slot])

        def g_wait(j, slot):
            pltpu.make_async_copy(
                hs_hbm.at[src_v.at[j]], rows_v.at[slot], gsem.at[slot]
            ).wait()

        def s_fire(j, slot):
            pltpu.async_copy(
                rows_v.at[slot], racc.at[dst_v.at[j]], ssem.at[slot], add=True
            )

        def s_wait(j, slot):
            pltpu.make_async_copy(
                rows_v.at[slot], racc.at[dst_v.at[j]], ssem.at[slot]
            ).wait()

        for b in range(LOOK):
            g_fire(b, b)

        @pl.loop(0, RPS, step=ZR)
        def _(b):
            pltpu.make_async_copy(zer_v, racc.at[pl.ds(sid * RPS + b, ZR)], zsem).wait()

        plsc.subcore_barrier()

        for b in range(NBUF):
            if b + LOOK < NBUF:
                g_fire(b + LOOK, b + LOOK)
            else:
                s_wait(b + LOOK - NBUF, (b + LOOK) % NBUF)
                g_fire(b + LOOK, (b + LOOK) % NBUF)
            g_wait(b, b)
            s_fire(b, b)

        @pl.loop(1, CHPW // NBUF - 1)
        def _(g):
            j0 = g * NBUF
            for b in range(NBUF):
                sl = (b + LOOK) % NBUF
                s_wait(j0 + b + LOOK - NBUF, sl)
                g_fire(j0 + b + LOOK, sl)
                g_wait(j0 + b, b)
                s_fire(j0 + b, b)

        j0 = CHPW - NBUF
        for b in range(NBUF):
            sl = (b + LOOK) % NBUF
            s_wait(j0 + b + LOOK - NBUF, sl)
            if b < NBUF - LOOK:
                g_fire(j0 + b + LOOK, sl)
            g_wait(j0 + b, b)
            s_fire(j0 + b, b)

        for b in range(LOOK):
            sl = (CHPW - LOOK + b) % NBUF
            s_wait(CHPW - LOOK + b, sl)

        plsc.subcore_barrier()
        pltpu.sync_copy(
            racc.at[pl.ds(sid * RPS, RPS)],
            out_hbm.at[cid, pl.ds(sid * RPS, RPS)],
        )

    return k(src3, dst3, hs2)


def kernel(x, edge_index, W1, b1, W2, b2):
    npad = EPAD - E
    pad_src = (jnp.arange(npad, dtype=jnp.int32) * 97) % N
    pad_dst = N + (jnp.arange(npad, dtype=jnp.int32) % (N2 - N))
    src3 = jnp.concatenate([edge_index[0], pad_src]).reshape(NW, CHPW, CHUNK)
    dst3 = jnp.concatenate([edge_index[1], pad_dst]).reshape(NW, CHPW, CHUNK)

    x2 = jnp.pad(x, ((0, N2 - N), (0, 0)))
    h0p = _mlp(x2, W1, b1, W2, b2)
    dp = _deg_sc(dst3)
    hs0, d2b, sqb = _prep(dp, h0p)

    d2b1 = d2b.reshape(_FLAT)
    hs01 = hs0.reshape(_FLAT)
    hs1 = hs01
    for step in range(K):
        p = _prop_sc(src3, dst3, hs1.reshape(N2, C))
        sq = sqb.reshape(_FLAT) if step == K - 1 else None
        hs1 = _combine(p.reshape(2 * _FLAT), hs1, d2b1, hs01, sq)
    return hs1.reshape(N2, C)[:N]
